# Initial kernel scaffold; baseline (speedup 1.0000x reference)
import jax
import jax.numpy as jnp
from jax.experimental import pallas as pl


def kernel(x1, x2, edge_index):
    raise NotImplementedError("write your pallas kernel here")


# speedup vs baseline: 2.1617x; 2.1617x over previous
import functools

import jax
import jax.numpy as jnp
from jax import lax
from jax.experimental import pallas as pl
from jax.experimental.pallas import tpu as pltpu
from jax.experimental.pallas import tpu_sc as plsc

NC = 2
NS = 16
L = 16

N_NODES = 10000
N_FEAT = 128
N_PAIR = N_FEAT // 2
E_TOTAL = 320000
EPW = E_TOTAL // (NC * NS)
CH = 400
N_CHUNK = EPW // CH
N_GROUP = CH // L


def _body(table, src, dst, out, idx_s, idx_d, rows_s, rows_d, out_v,
          sem_s, sem_d):
    wid = lax.axis_index("s") * NC + lax.axis_index("c")
    base_w = wid * EPW

    def chunk(ci, _):
        base = base_w + ci * CH
        pltpu.sync_copy(src.at[pl.ds(base, CH)], idx_s)
        pltpu.sync_copy(dst.at[pl.ds(base, CH)], idx_d)
        cp_s = pltpu.async_copy(table.at[idx_s], rows_s, sem_s)
        cp_d = pltpu.async_copy(table.at[idx_d], rows_d, sem_d)
        cp_s.wait()
        cp_d.wait()

        def group(g, _):
            row16 = g * L + lax.iota(jnp.int32, L)
            acc = jnp.zeros((L,), jnp.float32)
            for p in range(N_PAIR):
                col = jnp.full((L,), p, jnp.int32)
                a = plsc.load_gather(rows_s, [row16, col])
                b = plsc.load_gather(rows_d, [row16, col])
                a_lo = plsc.bitcast(lax.shift_left(a, 16), jnp.float32)
                b_lo = plsc.bitcast(lax.shift_left(b, 16), jnp.float32)
                a_hi = plsc.bitcast(a, jnp.float32)
                b_hi = plsc.bitcast(b, jnp.float32)
                acc = acc + a_lo * b_lo + a_hi * b_hi
            out_v[pl.ds(g * L, L)] = acc
            return _

        lax.fori_loop(0, N_GROUP, group, None)
        pltpu.sync_copy(out_v, out.at[pl.ds(base, CH)])
        return _

    lax.fori_loop(0, N_CHUNK, chunk, None)


@jax.jit
def kernel(x1, x2, edge_index):
    x = jnp.concatenate([x1, x2], axis=0).astype(jnp.bfloat16)
    table = lax.bitcast_convert_type(
        x.reshape(N_NODES, N_PAIR, 2), jnp.int32)
    src = edge_index[0].astype(jnp.int32)
    dst = edge_index[1].astype(jnp.int32)

    mesh = plsc.VectorSubcoreMesh(core_axis_name="c", subcore_axis_name="s")
    run = pl.kernel(
        _body,
        out_type=jax.ShapeDtypeStruct((E_TOTAL,), jnp.float32),
        mesh=mesh,
        compiler_params=pltpu.CompilerParams(
            needs_layout_passes=False, use_tc_tiling_on_sc=False),
        scratch_types=[
            pltpu.VMEM((CH,), jnp.int32),
            pltpu.VMEM((CH,), jnp.int32),
            pltpu.VMEM((CH, N_PAIR), jnp.int32),
            pltpu.VMEM((CH, N_PAIR), jnp.int32),
            pltpu.VMEM((CH,), jnp.float32),
            pltpu.SemaphoreType.DMA,
            pltpu.SemaphoreType.DMA,
        ],
    )
    return run(table, src, dst)


# speedup vs baseline: 7.1872x; 3.3248x over previous
import functools

import jax
import jax.numpy as jnp
from jax import lax
from jax.experimental import pallas as pl
from jax.experimental.pallas import tpu as pltpu
from jax.experimental.pallas import tpu_sc as plsc

NC = 2
NS = 16
L = 16

N_NODES = 10000
N_FEAT = 128
N_PAIR = N_FEAT // 2
E_TOTAL = 320000
EPW = E_TOTAL // (NC * NS)
CH = 400
N_CHUNK = EPW // CH
N_GROUP = CH // L


def _body(table, src, dst, out, idx_s, idx_d, rows_s, rows_d, out_v,
          sem_s, sem_d):
    wid = lax.axis_index("s") * NC + lax.axis_index("c")
    base_w = wid * EPW

    def chunk(ci, _):
        base = base_w + ci * CH
        pltpu.sync_copy(src.at[pl.ds(base, CH)], idx_s)
        pltpu.sync_copy(dst.at[pl.ds(base, CH)], idx_d)
        cp_s = pltpu.async_copy(table.at[idx_s], rows_s, sem_s)
        cp_d = pltpu.async_copy(table.at[idx_d], rows_d, sem_d)
        cp_s.wait()
        cp_d.wait()

        lanes = lax.iota(jnp.int32, L)

        def group(g, _):
            acc_out = jnp.zeros((L,), jnp.float32)
            for e in range(L):
                row = g * L + e
                acc = None
                for k in range(N_PAIR // L):
                    a = rows_s[row, pl.ds(k * L, L)]
                    b = rows_d[row, pl.ds(k * L, L)]
                    lo = (plsc.bitcast(lax.shift_left(a, 16), jnp.float32)
                          * plsc.bitcast(lax.shift_left(b, 16), jnp.float32))
                    hi = (plsc.bitcast(a, jnp.float32)
                          * plsc.bitcast(b, jnp.float32))
                    t = lo + hi
                    acc = t if acc is None else acc + t
                s = jnp.sum(acc)
                acc_out = jnp.where(lanes == e, s, acc_out)
            out_v[pl.ds(g * L, L)] = acc_out
            return _

        lax.fori_loop(0, N_GROUP, group, None)
        pltpu.sync_copy(out_v, out.at[pl.ds(base, CH)])
        return _

    lax.fori_loop(0, N_CHUNK, chunk, None)


@jax.jit
def kernel(x1, x2, edge_index):
    x = jnp.concatenate([x1, x2], axis=0).astype(jnp.bfloat16)
    table = lax.bitcast_convert_type(
        x.reshape(N_NODES, N_PAIR, 2), jnp.int32)
    src = edge_index[0].astype(jnp.int32)
    dst = edge_index[1].astype(jnp.int32)

    mesh = plsc.VectorSubcoreMesh(core_axis_name="c", subcore_axis_name="s")
    run = pl.kernel(
        _body,
        out_type=jax.ShapeDtypeStruct((E_TOTAL,), jnp.float32),
        mesh=mesh,
        compiler_params=pltpu.CompilerParams(
            needs_layout_passes=False, use_tc_tiling_on_sc=False),
        scratch_types=[
            pltpu.VMEM((CH,), jnp.int32),
            pltpu.VMEM((CH,), jnp.int32),
            pltpu.VMEM((CH, N_PAIR), jnp.int32),
            pltpu.VMEM((CH, N_PAIR), jnp.int32),
            pltpu.VMEM((CH,), jnp.float32),
            pltpu.SemaphoreType.DMA,
            pltpu.SemaphoreType.DMA,
        ],
    )
    return run(table, src, dst)


# speedup vs baseline: 9.5264x; 1.3255x over previous
import functools

import jax
import jax.numpy as jnp
from jax import lax
from jax.experimental import pallas as pl
from jax.experimental.pallas import tpu as pltpu
from jax.experimental.pallas import tpu_sc as plsc

NC = 2
NS = 16
L = 16

N_NODES = 10000
N_FEAT = 128
N_PAIR = N_FEAT // 2
E_TOTAL = 320000
EPW = E_TOTAL // (NC * NS)
CH = 80
N_CHUNK = EPW // CH
N_GROUP = CH // L


def _body(table, src, dst, out, idx_s, idx_d, out_v,
          rows_sa, rows_da, rows_sb, rows_db, sem_a, sem_b):
    wid = lax.axis_index("s") * NC + lax.axis_index("c")
    base_w = wid * EPW

    pltpu.sync_copy(src.at[pl.ds(base_w, EPW)], idx_s)
    pltpu.sync_copy(dst.at[pl.ds(base_w, EPW)], idx_d)

    def start_gather(c, rows_s_buf, rows_d_buf, sem):
        pltpu.async_copy(
            table.at[idx_s.at[pl.ds(c * CH, CH)]], rows_s_buf, sem)
        pltpu.async_copy(
            table.at[idx_d.at[pl.ds(c * CH, CH)]], rows_d_buf, sem)

    def wait_gather(rows_s_buf, rows_d_buf, sem):
        pltpu.make_async_copy(
            table.at[idx_s.at[pl.ds(0, CH)]], rows_s_buf, sem).wait()
        pltpu.make_async_copy(
            table.at[idx_d.at[pl.ds(0, CH)]], rows_d_buf, sem).wait()

    lanes = lax.iota(jnp.int32, L)

    def compute(c, rows_s_buf, rows_d_buf):
        def group(g, _):
            acc_out = jnp.zeros((L,), jnp.float32)
            for e in range(L):
                row = g * L + e
                acc = None
                for k in range(N_PAIR // L):
                    a = rows_s_buf[row, pl.ds(k * L, L)]
                    b = rows_d_buf[row, pl.ds(k * L, L)]
                    lo = (plsc.bitcast(lax.shift_left(a, 16), jnp.float32)
                          * plsc.bitcast(lax.shift_left(b, 16), jnp.float32))
                    hi = (plsc.bitcast(a, jnp.float32)
                          * plsc.bitcast(b, jnp.float32))
                    t = lo + hi
                    acc = t if acc is None else acc + t
                s = jnp.sum(acc)
                acc_out = jnp.where(lanes == e, s, acc_out)
            out_v[pl.ds(c * CH + g * L, L)] = acc_out
            return _

        lax.fori_loop(0, N_GROUP, group, None)

    start_gather(0, rows_sa, rows_da, sem_a)

    def pair(i, _):
        c0 = 2 * i
        start_gather(c0 + 1, rows_sb, rows_db, sem_b)
        wait_gather(rows_sa, rows_da, sem_a)
        compute(c0, rows_sa, rows_da)
        start_gather(c0 + 2, rows_sa, rows_da, sem_a)
        wait_gather(rows_sb, rows_db, sem_b)
        compute(c0 + 1, rows_sb, rows_db)
        return _

    lax.fori_loop(0, (N_CHUNK - 1) // 2, pair, None)

    wait_gather(rows_sa, rows_da, sem_a)
    compute(N_CHUNK - 1, rows_sa, rows_da)

    pltpu.sync_copy(out_v, out.at[pl.ds(base_w, EPW)])


@jax.jit
def kernel(x1, x2, edge_index):
    x = jnp.concatenate([x1, x2], axis=0).astype(jnp.bfloat16)
    table = lax.bitcast_convert_type(
        x.reshape(N_NODES, N_PAIR, 2), jnp.int32)
    src = edge_index[0].astype(jnp.int32)
    dst = edge_index[1].astype(jnp.int32)

    mesh = plsc.VectorSubcoreMesh(core_axis_name="c", subcore_axis_name="s")
    run = pl.kernel(
        _body,
        out_type=jax.ShapeDtypeStruct((E_TOTAL,), jnp.float32),
        mesh=mesh,
        compiler_params=pltpu.CompilerParams(
            needs_layout_passes=False, use_tc_tiling_on_sc=False),
        scratch_types=[
            pltpu.VMEM((EPW,), jnp.int32),
            pltpu.VMEM((EPW,), jnp.int32),
            pltpu.VMEM((EPW,), jnp.float32),
            pltpu.VMEM((CH, N_PAIR), jnp.int32),
            pltpu.VMEM((CH, N_PAIR), jnp.int32),
            pltpu.VMEM((CH, N_PAIR), jnp.int32),
            pltpu.VMEM((CH, N_PAIR), jnp.int32),
            pltpu.SemaphoreType.DMA,
            pltpu.SemaphoreType.DMA,
        ],
    )
    return run(table, src, dst)
